# Initial kernel scaffold; baseline (speedup 1.0000x reference)
import jax
import jax.numpy as jnp
from jax.experimental import pallas as pl


def kernel(class_labels, z, class_table, Wg1, bg1, Wg2, bg2, We, be, threshold):
    raise NotImplementedError("write your pallas kernel here")


# speedup vs baseline: 245.1433x; 245.1433x over previous
import functools

import numpy as np
import jax
import jax.numpy as jnp
from jax import lax
from jax.experimental import pallas as pl
from jax.experimental.pallas import tpu as pltpu
from jax.experimental.pallas import tpu_sc as plsc

_N = 2048
_NOISE_DIM = 128
_CLASS_EMBED_DIM = 64
_HIDDEN_DIM = 512
_NODE_FEAT_DIM = 256
_NUM_CLASSES = 10

_M = _N * (_N - 1) // 2
_MPAD = 1 << 21
_LANES = 128
_NW = 32
_PER_W = _MPAD // _NW
_CHUNK = 8192
_ROWS_PER_CHUNK = _CHUNK // _LANES
_NCHUNKS = _PER_W // _CHUNK
_BLK = 256

_iu_np, _ju_np = np.triu_indices(_N, k=1)
_FLAT_IDX_NP = np.zeros((_MPAD,), np.int32)
_FLAT_IDX_NP[:_M] = (_iu_np * _N + _ju_np).astype(np.int32)
_IDX2D_NP = _FLAT_IDX_NP.reshape(_MPAD // _LANES, _LANES)
_PAIR_NP = np.stack([_iu_np, _ju_np], axis=0).astype(np.int32)


def _mlp_body(labels_ref, ctab_ref, z_ref, w1z_ref, w1c_ref, b1_ref,
              w2_ref, b2_ref, we_ref, be_ref, h_ref):
    lab = labels_ref[0]
    sel = (lax.broadcasted_iota(jnp.int32, (_NUM_CLASSES, 1), 0) == lab)
    ce = jnp.sum(jnp.where(sel, ctab_ref[...], 0.0), axis=0, keepdims=True)
    b1_eff = b1_ref[...] + jnp.dot(ce, w1c_ref[...],
                                   preferred_element_type=jnp.float32)
    hgen = jnp.maximum(
        jnp.dot(z_ref[...], w1z_ref[...],
                preferred_element_type=jnp.float32) + b1_eff, 0.0)
    x = jnp.dot(hgen, w2_ref[...],
                preferred_element_type=jnp.float32) + b2_ref[...]
    h_ref[...] = jnp.maximum(
        jnp.dot(x, we_ref[...],
                preferred_element_type=jnp.float32) + be_ref[...], 0.0)


def _adj_body(thr_ref, hi_ref, h_ref, out_ref):
    i = pl.program_id(0)
    hi = hi_ref[...]
    h = h_ref[...]
    g = lax.dot_general(hi, h, (((1,), (1,)), ((), ())),
                        preferred_element_type=jnp.float32)
    sq_i = jnp.sum(hi * hi, axis=1, keepdims=True)
    sq_j = lax.dot_general(jnp.ones((1, _HIDDEN_DIM), jnp.float32), h * h,
                           (((1,), (1,)), ((), ())),
                           preferred_element_type=jnp.float32)
    d2 = sq_i + sq_j - 2.0 * g
    dist = jnp.sqrt(jnp.clip(d2, 1e-12, None))
    probs = jax.nn.sigmoid(thr_ref[0] - dist)
    rows = i * _BLK + lax.broadcasted_iota(jnp.int32, (_BLK, _N), 0)
    cols = lax.broadcasted_iota(jnp.int32, (_BLK, _N), 1)
    out_ref[...] = jnp.where(rows == cols, 0.0, probs)


@functools.cache
def _make_triu_gather():
    @functools.partial(
        pl.kernel,
        out_type=jax.ShapeDtypeStruct((_MPAD,), jnp.float32),
        mesh=plsc.VectorSubcoreMesh(core_axis_name="c", subcore_axis_name="s"),
        scratch_types=[
            pltpu.VMEM((_CHUNK,), jnp.int32),
            pltpu.VMEM((_CHUNK,), jnp.float32),
            pltpu.SemaphoreType.DMA,
        ],
    )
    def _triu_gather(adj_hbm, idx_hbm, out_hbm, idx_v, val_v, sem):
        cid = lax.axis_index("c")
        sid = lax.axis_index("s")
        wid = sid * 2 + cid

        def chunk(k, carry):
            base = wid * _PER_W + k * _CHUNK
            pltpu.sync_copy(idx_hbm.at[pl.ds(base, _CHUNK)], idx_v)
            pltpu.async_copy(adj_hbm.at[idx_v], val_v, sem).wait()
            pltpu.sync_copy(val_v, out_hbm.at[pl.ds(base, _CHUNK)])
            return carry

        lax.fori_loop(0, _NCHUNKS, chunk, 0)

    return _triu_gather


def kernel(class_labels, z, class_table, Wg1, bg1, Wg2, bg2, We, be, threshold):
    w1z = Wg1[:_NOISE_DIM]
    w1c = Wg1[_NOISE_DIM:]

    h = pl.pallas_call(
        _mlp_body,
        out_shape=jax.ShapeDtypeStruct((_N, _HIDDEN_DIM), jnp.float32),
        in_specs=[
            pl.BlockSpec(memory_space=pltpu.SMEM),
            pl.BlockSpec(memory_space=pltpu.VMEM),
            pl.BlockSpec(memory_space=pltpu.VMEM),
            pl.BlockSpec(memory_space=pltpu.VMEM),
            pl.BlockSpec(memory_space=pltpu.VMEM),
            pl.BlockSpec(memory_space=pltpu.VMEM),
            pl.BlockSpec(memory_space=pltpu.VMEM),
            pl.BlockSpec(memory_space=pltpu.VMEM),
            pl.BlockSpec(memory_space=pltpu.VMEM),
            pl.BlockSpec(memory_space=pltpu.VMEM),
        ],
        out_specs=pl.BlockSpec(memory_space=pltpu.VMEM),
    )(class_labels, class_table, z, w1z, w1c, bg1[None, :], Wg2,
      bg2[None, :], We, be[None, :])

    soft_adj = pl.pallas_call(
        _adj_body,
        grid=(_N // _BLK,),
        out_shape=jax.ShapeDtypeStruct((_N, _N), jnp.float32),
        in_specs=[
            pl.BlockSpec(memory_space=pltpu.SMEM),
            pl.BlockSpec((_BLK, _HIDDEN_DIM), lambda i: (i, 0)),
            pl.BlockSpec((_N, _HIDDEN_DIM), lambda i: (0, 0)),
        ],
        out_specs=pl.BlockSpec((_BLK, _N), lambda i: (i, 0)),
    )(jnp.reshape(threshold, (1,)), h, h)

    idx = jnp.asarray(_FLAT_IDX_NP)
    probs_pad = _make_triu_gather()(jnp.reshape(soft_adj, (_N * _N,)), idx)
    probs = probs_pad[:_M][:, None]

    pair_index = jnp.asarray(_PAIR_NP)
    return probs, pair_index, soft_adj


# speedup vs baseline: 436.4076x; 1.7802x over previous
import functools

import numpy as np
import jax
import jax.numpy as jnp
from jax import lax
from jax.experimental import pallas as pl
from jax.experimental.pallas import tpu as pltpu
from jax.experimental.pallas import tpu_sc as plsc

_N = 2048
_NOISE_DIM = 128
_CLASS_EMBED_DIM = 64
_HIDDEN_DIM = 512
_NODE_FEAT_DIM = 256
_NUM_CLASSES = 10

_M = _N * (_N - 1) // 2
_NW = 32
_SEG = _M // _NW
_KBUF = 8
_ROWP = 2064
_BLK = 256

_iu_np, _ju_np = np.triu_indices(_N, k=1)
_PAIR_NP = np.stack([_iu_np, _ju_np], axis=0).astype(np.int32)

_OFF_NP = (np.arange(_N + 1, dtype=np.int64) *
           (2 * _N - 1 - np.arange(_N + 1, dtype=np.int64))) // 2
_R0S = [int(np.searchsorted(_OFF_NP, w * _SEG, side="right") - 1)
        for w in range(_NW)]
_R1S = [int(np.searchsorted(_OFF_NP, w * _SEG + _SEG - 1, side="right"))
        for w in range(_NW)]


def _mlp_body(labels_ref, ctab_ref, z_ref, w1z_ref, w1c_ref, b1_ref,
              w2_ref, b2_ref, we_ref, be_ref, h_ref):
    lab = labels_ref[0]
    sel = (lax.broadcasted_iota(jnp.int32, (_NUM_CLASSES, 1), 0) == lab)
    ce = jnp.sum(jnp.where(sel, ctab_ref[...], 0.0), axis=0, keepdims=True)
    b1_eff = b1_ref[...] + jnp.dot(ce, w1c_ref[...],
                                   preferred_element_type=jnp.float32)
    hgen = jnp.maximum(
        jnp.dot(z_ref[...], w1z_ref[...],
                preferred_element_type=jnp.float32) + b1_eff, 0.0)
    x = jnp.dot(hgen, w2_ref[...],
                preferred_element_type=jnp.float32) + b2_ref[...]
    h_ref[...] = jnp.maximum(
        jnp.dot(x, we_ref[...],
                preferred_element_type=jnp.float32) + be_ref[...], 0.0)


def _adj_body(thr_ref, hi_ref, h_ref, out_ref):
    i = pl.program_id(0)
    hi = hi_ref[...]
    h = h_ref[...]
    g = lax.dot_general(hi, h, (((1,), (1,)), ((), ())),
                        preferred_element_type=jnp.float32)
    sq_i = jnp.sum(hi * hi, axis=1, keepdims=True)
    sq_j = lax.dot_general(jnp.ones((1, _HIDDEN_DIM), jnp.float32), h * h,
                           (((1,), (1,)), ((), ())),
                           preferred_element_type=jnp.float32)
    d2 = sq_i + sq_j - 2.0 * g
    dist = jnp.sqrt(jnp.clip(d2, 1e-12, None))
    probs = jax.nn.sigmoid(thr_ref[0] - dist)
    rows = i * _BLK + lax.broadcasted_iota(jnp.int32, (_BLK, _N), 0)
    cols = lax.broadcasted_iota(jnp.int32, (_BLK, _N), 1)
    out_ref[...] = jnp.where(rows == cols, 0.0, probs)


def _tri_off(i):
    return (i * (2 * _N - 1 - i)) // 2


@functools.cache
def _make_triu_gather():
    @functools.partial(
        pl.kernel,
        out_type=jax.ShapeDtypeStruct((_M,), jnp.float32),
        mesh=plsc.VectorSubcoreMesh(core_axis_name="c", subcore_axis_name="s"),
        scratch_types=[
            pltpu.VMEM((2 * _KBUF * _ROWP,), jnp.float32),
            pltpu.VMEM((_SEG + _ROWP,), jnp.float32),
            [pltpu.SemaphoreType.DMA] * (2 * _KBUF),
        ],
    )
    def _triu_gather(adj_hbm, out_hbm, rows_v, seg_v, sems):
        cid = lax.axis_index("c")
        sid = lax.axis_index("s")
        wid = sid * 2 + cid
        a0 = wid * _SEG

        r0 = jnp.int32(_R0S[0])
        r1 = jnp.int32(_R1S[0])
        for w in range(1, _NW):
            r0 = jnp.where(wid == w, jnp.int32(_R0S[w]), r0)
            r1 = jnp.where(wid == w, jnp.int32(_R1S[w]), r1)
        nrows = r1 - r0
        ngroups = (nrows + _KBUF - 1) // _KBUF

        def fire(t, bank):
            rbase = r0 + t * _KBUF
            for b in range(_KBUF):
                i = rbase + b
                slot = bank * _KBUF + b

                @pl.when((t < ngroups) & (i < r1))
                def _():
                    pltpu.async_copy(
                        adj_hbm.at[pl.ds(i * _N, _N)],
                        rows_v.at[pl.ds(slot * _ROWP, _N)],
                        sems[slot])

        def process(t, bank):
            rbase = r0 + t * _KBUF
            for b in range(_KBUF):
                i = rbase + b
                slot = bank * _KBUF + b

                @pl.when((t < ngroups) & (i < r1))
                def _():
                    pltpu.make_async_copy(
                        adj_hbm.at[pl.ds(i * _N, _N)],
                        rows_v.at[pl.ds(slot * _ROWP, _N)],
                        sems[slot]).wait()
                    off_i = _tri_off(i)
                    skip = jnp.maximum(a0 - off_i, 0)
                    col0 = i + 1 + skip
                    q = off_i + skip - a0
                    length = (_N - 1 - i) - skip
                    nv = (length + 15) >> 4
                    src0 = slot * _ROWP + col0

                    def copy16(u, carry):
                        seg_v[pl.ds(q + u * 16, 16)] = (
                            rows_v[pl.ds(src0 + u * 16, 16)])
                        return carry

                    lax.fori_loop(0, nv, copy16, 0)

        fire(0, 0)

        def two_groups(tt, carry):
            t0 = 2 * tt
            fire(t0 + 1, 1)
            process(t0, 0)
            fire(t0 + 2, 0)
            process(t0 + 1, 1)
            return carry

        lax.fori_loop(0, (ngroups + 1) // 2, two_groups, 0)
        pltpu.sync_copy(seg_v.at[pl.ds(0, _SEG)], out_hbm.at[pl.ds(a0, _SEG)])

    return _triu_gather


def kernel(class_labels, z, class_table, Wg1, bg1, Wg2, bg2, We, be, threshold):
    w1z = Wg1[:_NOISE_DIM]
    w1c = Wg1[_NOISE_DIM:]

    h = pl.pallas_call(
        _mlp_body,
        out_shape=jax.ShapeDtypeStruct((_N, _HIDDEN_DIM), jnp.float32),
        in_specs=[
            pl.BlockSpec(memory_space=pltpu.SMEM),
            pl.BlockSpec(memory_space=pltpu.VMEM),
            pl.BlockSpec(memory_space=pltpu.VMEM),
            pl.BlockSpec(memory_space=pltpu.VMEM),
            pl.BlockSpec(memory_space=pltpu.VMEM),
            pl.BlockSpec(memory_space=pltpu.VMEM),
            pl.BlockSpec(memory_space=pltpu.VMEM),
            pl.BlockSpec(memory_space=pltpu.VMEM),
            pl.BlockSpec(memory_space=pltpu.VMEM),
            pl.BlockSpec(memory_space=pltpu.VMEM),
        ],
        out_specs=pl.BlockSpec(memory_space=pltpu.VMEM),
    )(class_labels, class_table, z, w1z, w1c, bg1[None, :], Wg2,
      bg2[None, :], We, be[None, :])

    soft_adj = pl.pallas_call(
        _adj_body,
        grid=(_N // _BLK,),
        out_shape=jax.ShapeDtypeStruct((_N, _N), jnp.float32),
        in_specs=[
            pl.BlockSpec(memory_space=pltpu.SMEM),
            pl.BlockSpec((_BLK, _HIDDEN_DIM), lambda i: (i, 0)),
            pl.BlockSpec((_N, _HIDDEN_DIM), lambda i: (0, 0)),
        ],
        out_specs=pl.BlockSpec((_BLK, _N), lambda i: (i, 0)),
    )(jnp.reshape(threshold, (1,)), h, h)

    probs_flat = _make_triu_gather()(jnp.reshape(soft_adj, (_N * _N,)))
    probs = probs_flat[:, None]

    pair_index = jnp.asarray(_PAIR_NP)
    return probs, pair_index, soft_adj


# speedup vs baseline: 470.4419x; 1.0780x over previous
import functools

import numpy as np
import jax
import jax.numpy as jnp
from jax import lax
from jax.experimental import pallas as pl
from jax.experimental.pallas import tpu as pltpu
from jax.experimental.pallas import tpu_sc as plsc

_N = 2048
_NOISE_DIM = 128
_CLASS_EMBED_DIM = 64
_HIDDEN_DIM = 512
_NODE_FEAT_DIM = 256
_NUM_CLASSES = 10

_M = _N * (_N - 1) // 2
_NW = 32
_SEG = _M // _NW
_KBUF = 8
_ROWP = 2064
_BLK = 256

_iu_np, _ju_np = np.triu_indices(_N, k=1)
_PAIR_NP = np.stack([_iu_np, _ju_np], axis=0).astype(np.int32)

_OFF_NP = (np.arange(_N + 1, dtype=np.int64) *
           (2 * _N - 1 - np.arange(_N + 1, dtype=np.int64))) // 2
_R0S = [int(np.searchsorted(_OFF_NP, w * _SEG, side="right") - 1)
        for w in range(_NW)]
_R1S = [int(np.searchsorted(_OFF_NP, w * _SEG + _SEG - 1, side="right"))
        for w in range(_NW)]
_WCLS = (2048, 1024, 512)
_CLS = [max(c for c, wdt in enumerate(_WCLS) if wdt >= 2047 - _R0S[w])
        for w in range(_NW)]


def _mlp_body(labels_ref, ctab_ref, z_ref, w1z_ref, w1c_ref, b1_ref,
              w2_ref, b2_ref, we_ref, be_ref, h_ref):
    lab = labels_ref[0]
    sel = (lax.broadcasted_iota(jnp.int32, (_NUM_CLASSES, 1), 0) == lab)
    ce = jnp.sum(jnp.where(sel, ctab_ref[...], 0.0), axis=0, keepdims=True)
    b1_eff = b1_ref[...] + jnp.dot(ce, w1c_ref[...],
                                   preferred_element_type=jnp.float32)
    hgen = jnp.maximum(
        jnp.dot(z_ref[...], w1z_ref[...],
                preferred_element_type=jnp.float32) + b1_eff, 0.0)
    x = jnp.dot(hgen, w2_ref[...],
                preferred_element_type=jnp.float32) + b2_ref[...]
    h_ref[...] = jnp.maximum(
        jnp.dot(x, we_ref[...],
                preferred_element_type=jnp.float32) + be_ref[...], 0.0)


def _adj_body(thr_ref, hi_ref, h_ref, out_ref):
    i = pl.program_id(0)
    hi = hi_ref[...]
    h = h_ref[...]
    g = lax.dot_general(hi, h, (((1,), (1,)), ((), ())),
                        preferred_element_type=jnp.float32)
    sq_i = jnp.sum(hi * hi, axis=1, keepdims=True)
    sq_j = lax.dot_general(jnp.ones((1, _HIDDEN_DIM), jnp.float32), h * h,
                           (((1,), (1,)), ((), ())),
                           preferred_element_type=jnp.float32)
    d2 = sq_i + sq_j - 2.0 * g
    dist = jnp.sqrt(jnp.clip(d2, 1e-12, None))
    probs = jax.nn.sigmoid(thr_ref[0] - dist)
    rows = i * _BLK + lax.broadcasted_iota(jnp.int32, (_BLK, _N), 0)
    cols = lax.broadcasted_iota(jnp.int32, (_BLK, _N), 1)
    out_ref[...] = jnp.where(rows == cols, 0.0, probs)


def _tri_off(i):
    return (i * (2 * _N - 1 - i)) // 2


@functools.cache
def _make_triu_gather():
    @functools.partial(
        pl.kernel,
        out_type=jax.ShapeDtypeStruct((_M,), jnp.float32),
        mesh=plsc.VectorSubcoreMesh(core_axis_name="c", subcore_axis_name="s"),
        scratch_types=[
            pltpu.VMEM((2 * _KBUF * _ROWP,), jnp.float32),
            pltpu.VMEM((_SEG + _ROWP,), jnp.float32),
            [pltpu.SemaphoreType.DMA] * (2 * _KBUF),
        ],
    )
    def _triu_gather(adj_hbm, out_hbm, rows_v, seg_v, sems):
        cid = lax.axis_index("c")
        sid = lax.axis_index("s")
        wid = sid * 2 + cid
        a0 = wid * _SEG

        r0 = jnp.int32(_R0S[0])
        r1 = jnp.int32(_R1S[0])
        cls = jnp.int32(_CLS[0])
        for w in range(1, _NW):
            r0 = jnp.where(wid == w, jnp.int32(_R0S[w]), r0)
            r1 = jnp.where(wid == w, jnp.int32(_R1S[w]), r1)
            cls = jnp.where(wid == w, jnp.int32(_CLS[w]), cls)
        nrows = r1 - r0
        wsel = jnp.int32(_WCLS[0])
        for c in range(1, len(_WCLS)):
            wsel = jnp.where(cls == c, jnp.int32(_WCLS[c]), wsel)
        ngroups = (nrows + _KBUF - 1) // _KBUF

        def fire(t, bank):
            rbase = r0 + t * _KBUF
            for b in range(_KBUF):
                i = rbase + b
                slot = bank * _KBUF + b
                live = (t < ngroups) & (i < r1)
                for c, wdt in enumerate(_WCLS):

                    @pl.when(live & (cls == c))
                    def _(i=i, slot=slot, wdt=wdt):
                        pltpu.async_copy(
                            adj_hbm.at[pl.ds(i * _N + (_N - wdt), wdt)],
                            rows_v.at[pl.ds(slot * _ROWP, wdt)],
                            sems[slot])

        def process(t, bank):
            rbase = r0 + t * _KBUF
            for b in range(_KBUF):
                i = rbase + b
                slot = bank * _KBUF + b
                live = (t < ngroups) & (i < r1)
                for c, wdt in enumerate(_WCLS):

                    @pl.when(live & (cls == c))
                    def _(i=i, slot=slot, wdt=wdt):
                        pltpu.make_async_copy(
                            adj_hbm.at[pl.ds(i * _N + (_N - wdt), wdt)],
                            rows_v.at[pl.ds(slot * _ROWP, wdt)],
                            sems[slot]).wait()

                @pl.when(live)
                def _(i=i, slot=slot):
                    off_i = _tri_off(i)
                    skip = jnp.maximum(a0 - off_i, 0)
                    col0 = i + 1 + skip
                    q = off_i + skip - a0
                    length = (_N - 1 - i) - skip
                    nv = (length + 15) >> 4
                    src0 = slot * _ROWP + col0 - _N + wsel

                    def copy16(u, carry):
                        seg_v[pl.ds(q + u * 16, 16)] = (
                            rows_v[pl.ds(src0 + u * 16, 16)])
                        return carry

                    lax.fori_loop(0, nv, copy16, 0)

        fire(0, 0)

        def two_groups(tt, carry):
            t0 = 2 * tt
            fire(t0 + 1, 1)
            process(t0, 0)
            fire(t0 + 2, 0)
            process(t0 + 1, 1)
            return carry

        lax.fori_loop(0, (ngroups + 1) // 2, two_groups, 0)
        pltpu.sync_copy(seg_v.at[pl.ds(0, _SEG)], out_hbm.at[pl.ds(a0, _SEG)])

    return _triu_gather


def kernel(class_labels, z, class_table, Wg1, bg1, Wg2, bg2, We, be, threshold):
    w1z = Wg1[:_NOISE_DIM]
    w1c = Wg1[_NOISE_DIM:]

    h = pl.pallas_call(
        _mlp_body,
        out_shape=jax.ShapeDtypeStruct((_N, _HIDDEN_DIM), jnp.float32),
        in_specs=[
            pl.BlockSpec(memory_space=pltpu.SMEM),
            pl.BlockSpec(memory_space=pltpu.VMEM),
            pl.BlockSpec(memory_space=pltpu.VMEM),
            pl.BlockSpec(memory_space=pltpu.VMEM),
            pl.BlockSpec(memory_space=pltpu.VMEM),
            pl.BlockSpec(memory_space=pltpu.VMEM),
            pl.BlockSpec(memory_space=pltpu.VMEM),
            pl.BlockSpec(memory_space=pltpu.VMEM),
            pl.BlockSpec(memory_space=pltpu.VMEM),
            pl.BlockSpec(memory_space=pltpu.VMEM),
        ],
        out_specs=pl.BlockSpec(memory_space=pltpu.VMEM),
    )(class_labels, class_table, z, w1z, w1c, bg1[None, :], Wg2,
      bg2[None, :], We, be[None, :])

    soft_adj = pl.pallas_call(
        _adj_body,
        grid=(_N // _BLK,),
        out_shape=jax.ShapeDtypeStruct((_N, _N), jnp.float32),
        in_specs=[
            pl.BlockSpec(memory_space=pltpu.SMEM),
            pl.BlockSpec((_BLK, _HIDDEN_DIM), lambda i: (i, 0)),
            pl.BlockSpec((_N, _HIDDEN_DIM), lambda i: (0, 0)),
        ],
        out_specs=pl.BlockSpec((_BLK, _N), lambda i: (i, 0)),
    )(jnp.reshape(threshold, (1,)), h, h)

    probs_flat = _make_triu_gather()(jnp.reshape(soft_adj, (_N * _N,)))
    probs = probs_flat[:, None]

    pair_index = jnp.asarray(_PAIR_NP)
    return probs, pair_index, soft_adj


# speedup vs baseline: 488.0863x; 1.0375x over previous
import functools

import numpy as np
import jax
import jax.numpy as jnp
from jax import lax
from jax.experimental import pallas as pl
from jax.experimental.pallas import tpu as pltpu
from jax.experimental.pallas import tpu_sc as plsc

_N = 2048
_NOISE_DIM = 128
_CLASS_EMBED_DIM = 64
_HIDDEN_DIM = 512
_NODE_FEAT_DIM = 256
_NUM_CLASSES = 10

_M = _N * (_N - 1) // 2
_NW = 32
_SEG = _M // _NW
_KBUF = 8
_ROWP = 2064
_BLK = 256

_iu_np, _ju_np = np.triu_indices(_N, k=1)
_PAIR_NP = np.stack([_iu_np, _ju_np], axis=0).astype(np.int32)

_OFF_NP = (np.arange(_N + 1, dtype=np.int64) *
           (2 * _N - 1 - np.arange(_N + 1, dtype=np.int64))) // 2
_R0S = [int(np.searchsorted(_OFF_NP, w * _SEG, side="right") - 1)
        for w in range(_NW)]
_R1S = [int(np.searchsorted(_OFF_NP, w * _SEG + _SEG - 1, side="right"))
        for w in range(_NW)]
_WCLS = (2048, 1024, 512)
_CLS = [max(c for c, wdt in enumerate(_WCLS) if wdt >= 2047 - _R0S[w])
        for w in range(_NW)]


def _gen_body(labels_ref, thr_ref, ctab_ref, z_ref, w1z_ref, w1c_ref, b1_ref,
              w2_ref, b2_ref, we_ref, be_ref, out_ref, h_scr):
    i = pl.program_id(0)

    @pl.when(i == 0)
    def _():
        lab = labels_ref[0]
        sel = (lax.broadcasted_iota(jnp.int32, (_NUM_CLASSES, 1), 0) == lab)
        ce = jnp.sum(jnp.where(sel, ctab_ref[...], 0.0), axis=0, keepdims=True)
        b1_eff = b1_ref[...] + jnp.dot(ce, w1c_ref[...],
                                       preferred_element_type=jnp.float32)
        hgen = jnp.maximum(
            jnp.dot(z_ref[...], w1z_ref[...],
                    preferred_element_type=jnp.float32) + b1_eff, 0.0)
        x = jnp.dot(hgen, w2_ref[...],
                    preferred_element_type=jnp.float32) + b2_ref[...]
        h_scr[...] = jnp.maximum(
            jnp.dot(x, we_ref[...],
                    preferred_element_type=jnp.float32) + be_ref[...], 0.0)

    hi = h_scr[pl.ds(i * _BLK, _BLK), :]
    h = h_scr[...]
    g = lax.dot_general(hi, h, (((1,), (1,)), ((), ())),
                        preferred_element_type=jnp.float32)
    sq_i = jnp.sum(hi * hi, axis=1, keepdims=True)
    sq_j = lax.dot_general(jnp.ones((1, _HIDDEN_DIM), jnp.float32), h * h,
                           (((1,), (1,)), ((), ())),
                           preferred_element_type=jnp.float32)
    d2 = sq_i + sq_j - 2.0 * g
    dist = jnp.sqrt(jnp.clip(d2, 1e-12, None))
    probs = jax.nn.sigmoid(thr_ref[0] - dist)
    rows = i * _BLK + lax.broadcasted_iota(jnp.int32, (_BLK, _N), 0)
    cols = lax.broadcasted_iota(jnp.int32, (_BLK, _N), 1)
    out_ref[...] = jnp.where(rows == cols, 0.0, probs)


def _tri_off(i):
    return (i * (2 * _N - 1 - i)) // 2


@functools.cache
def _make_triu_gather():
    @functools.partial(
        pl.kernel,
        out_type=jax.ShapeDtypeStruct((_M,), jnp.float32),
        mesh=plsc.VectorSubcoreMesh(core_axis_name="c", subcore_axis_name="s"),
        scratch_types=[
            pltpu.VMEM((2 * _KBUF * _ROWP,), jnp.float32),
            pltpu.VMEM((_SEG + _ROWP,), jnp.float32),
            [pltpu.SemaphoreType.DMA] * (2 * _KBUF),
        ],
    )
    def _triu_gather(adj_hbm, out_hbm, rows_v, seg_v, sems):
        cid = lax.axis_index("c")
        sid = lax.axis_index("s")
        wid = sid * 2 + cid
        a0 = wid * _SEG

        r0 = jnp.int32(_R0S[0])
        r1 = jnp.int32(_R1S[0])
        cls = jnp.int32(_CLS[0])
        for w in range(1, _NW):
            r0 = jnp.where(wid == w, jnp.int32(_R0S[w]), r0)
            r1 = jnp.where(wid == w, jnp.int32(_R1S[w]), r1)
            cls = jnp.where(wid == w, jnp.int32(_CLS[w]), cls)
        nrows = r1 - r0
        wsel = jnp.int32(_WCLS[0])
        for c in range(1, len(_WCLS)):
            wsel = jnp.where(cls == c, jnp.int32(_WCLS[c]), wsel)
        ngroups = (nrows + _KBUF - 1) // _KBUF

        def fire(t, bank):
            rbase = r0 + t * _KBUF
            for b in range(_KBUF):
                i = rbase + b
                slot = bank * _KBUF + b
                live = (t < ngroups) & (i < r1)
                for c, wdt in enumerate(_WCLS):

                    @pl.when(live & (cls == c))
                    def _(i=i, slot=slot, wdt=wdt):
                        pltpu.async_copy(
                            adj_hbm.at[pl.ds(i * _N + (_N - wdt), wdt)],
                            rows_v.at[pl.ds(slot * _ROWP, wdt)],
                            sems[slot])

        def process(t, bank):
            rbase = r0 + t * _KBUF
            for b in range(_KBUF):
                i = rbase + b
                slot = bank * _KBUF + b
                live = (t < ngroups) & (i < r1)
                for c, wdt in enumerate(_WCLS):

                    @pl.when(live & (cls == c))
                    def _(i=i, slot=slot, wdt=wdt):
                        pltpu.make_async_copy(
                            adj_hbm.at[pl.ds(i * _N + (_N - wdt), wdt)],
                            rows_v.at[pl.ds(slot * _ROWP, wdt)],
                            sems[slot]).wait()

                @pl.when(live)
                def _(i=i, slot=slot):
                    off_i = _tri_off(i)
                    skip = jnp.maximum(a0 - off_i, 0)
                    col0 = i + 1 + skip
                    q = off_i + skip - a0
                    length = (_N - 1 - i) - skip
                    nv = (length + 15) >> 4
                    src0 = slot * _ROWP + col0 - _N + wsel

                    def copy16(u, carry):
                        seg_v[pl.ds(q + u * 16, 16)] = (
                            rows_v[pl.ds(src0 + u * 16, 16)])
                        return carry

                    lax.fori_loop(0, nv, copy16, 0)

        fire(0, 0)

        def two_groups(tt, carry):
            t0 = 2 * tt
            fire(t0 + 1, 1)
            process(t0, 0)
            fire(t0 + 2, 0)
            process(t0 + 1, 1)
            return carry

        lax.fori_loop(0, (ngroups + 1) // 2, two_groups, 0)
        pltpu.sync_copy(seg_v.at[pl.ds(0, _SEG)], out_hbm.at[pl.ds(a0, _SEG)])

    return _triu_gather


def kernel(class_labels, z, class_table, Wg1, bg1, Wg2, bg2, We, be, threshold):
    w1z = Wg1[:_NOISE_DIM]
    w1c = Wg1[_NOISE_DIM:]

    soft_adj = pl.pallas_call(
        _gen_body,
        grid=(_N // _BLK,),
        out_shape=jax.ShapeDtypeStruct((_N, _N), jnp.float32),
        in_specs=[
            pl.BlockSpec(memory_space=pltpu.SMEM),
            pl.BlockSpec(memory_space=pltpu.SMEM),
            pl.BlockSpec(memory_space=pltpu.VMEM),
            pl.BlockSpec(memory_space=pltpu.VMEM),
            pl.BlockSpec(memory_space=pltpu.VMEM),
            pl.BlockSpec(memory_space=pltpu.VMEM),
            pl.BlockSpec(memory_space=pltpu.VMEM),
            pl.BlockSpec(memory_space=pltpu.VMEM),
            pl.BlockSpec(memory_space=pltpu.VMEM),
            pl.BlockSpec(memory_space=pltpu.VMEM),
            pl.BlockSpec(memory_space=pltpu.VMEM),
        ],
        out_specs=pl.BlockSpec((_BLK, _N), lambda i: (i, 0)),
        scratch_shapes=[pltpu.VMEM((_N, _HIDDEN_DIM), jnp.float32)],
    )(class_labels, jnp.reshape(threshold, (1,)), class_table, z, w1z, w1c,
      bg1[None, :], Wg2, bg2[None, :], We, be[None, :])

    probs_flat = _make_triu_gather()(jnp.reshape(soft_adj, (_N * _N,)))
    probs = probs_flat[:, None]
    pair_index = jnp.asarray(_PAIR_NP)
    return probs, pair_index, soft_adj


# speedup vs baseline: 546.7409x; 1.1202x over previous
import functools

import numpy as np
import jax
import jax.numpy as jnp
from jax import lax
from jax.experimental import pallas as pl
from jax.experimental.pallas import tpu as pltpu
from jax.experimental.pallas import tpu_sc as plsc

_N = 2048
_NOISE_DIM = 128
_CLASS_EMBED_DIM = 64
_HIDDEN_DIM = 512
_NODE_FEAT_DIM = 256
_NUM_CLASSES = 10

_M = _N * (_N - 1) // 2
_NW = 32
_SEG = _M // _NW
_KBUF = 8
_ROWP = 2064
_BLK = 256

_iu_np, _ju_np = np.triu_indices(_N, k=1)
_PAIR_NP = np.stack([_iu_np, _ju_np], axis=0).astype(np.int32)

_OFF_NP = (np.arange(_N + 1, dtype=np.int64) *
           (2 * _N - 1 - np.arange(_N + 1, dtype=np.int64))) // 2
_R0S = [int(np.searchsorted(_OFF_NP, w * _SEG, side="right") - 1)
        for w in range(_NW)]
_R1S = [int(np.searchsorted(_OFF_NP, w * _SEG + _SEG - 1, side="right"))
        for w in range(_NW)]
_WCLS = (2048, 1024, 512)
_CLS = [max(c for c, wdt in enumerate(_WCLS) if wdt >= 2047 - _R0S[w])
        for w in range(_NW)]


def _gen_body(labels_ref, thr_ref, ctab_ref, z_ref, w1_ref, b1_ref,
              w2_ref, b2_ref, we_ref, be_ref, pair_in_ref,
              out_ref, pair_out_ref, h_scr):
    i = pl.program_id(0)

    pair_out_ref[...] = pair_in_ref[...]

    @pl.when(i == 0)
    def _():
        lab = labels_ref[0]
        sel = (lax.broadcasted_iota(jnp.int32, (_NUM_CLASSES, 1), 0) == lab)
        ce = jnp.sum(jnp.where(sel, ctab_ref[...], 0.0), axis=0, keepdims=True)
        b1_eff = b1_ref[...] + jnp.dot(ce, w1_ref[pl.ds(_NOISE_DIM, _CLASS_EMBED_DIM), :],
                                       preferred_element_type=jnp.float32)
        hgen = jnp.maximum(
            jnp.dot(z_ref[...], w1_ref[pl.ds(0, _NOISE_DIM), :],
                    preferred_element_type=jnp.float32) + b1_eff, 0.0)
        x = jnp.dot(hgen, w2_ref[...],
                    preferred_element_type=jnp.float32) + b2_ref[...]
        h_scr[...] = jnp.maximum(
            jnp.dot(x, we_ref[...],
                    preferred_element_type=jnp.float32) + be_ref[...], 0.0)

    hi = h_scr[pl.ds(i * _BLK, _BLK), :]
    h = h_scr[...]
    g = lax.dot_general(hi, h, (((1,), (1,)), ((), ())),
                        preferred_element_type=jnp.float32)
    sq_i = jnp.sum(hi * hi, axis=1, keepdims=True)
    sq_j = lax.dot_general(jnp.ones((1, _HIDDEN_DIM), jnp.float32), h * h,
                           (((1,), (1,)), ((), ())),
                           preferred_element_type=jnp.float32)
    d2 = sq_i + sq_j - 2.0 * g
    dist = jnp.sqrt(jnp.clip(d2, 1e-12, None))
    probs = jax.nn.sigmoid(thr_ref[0] - dist)
    rows = i * _BLK + lax.broadcasted_iota(jnp.int32, (_BLK, _N), 0)
    cols = lax.broadcasted_iota(jnp.int32, (_BLK, _N), 1)
    out_ref[...] = jnp.where(rows == cols, 0.0, probs)


def _tri_off(i):
    return (i * (2 * _N - 1 - i)) // 2


@functools.cache
def _make_triu_gather():
    @functools.partial(
        pl.kernel,
        out_type=jax.ShapeDtypeStruct((_M,), jnp.float32),
        mesh=plsc.VectorSubcoreMesh(core_axis_name="c", subcore_axis_name="s"),
        scratch_types=[
            pltpu.VMEM((2 * _KBUF * _ROWP,), jnp.float32),
            pltpu.VMEM((_SEG + _ROWP,), jnp.float32),
            [pltpu.SemaphoreType.DMA] * (2 * _KBUF),
        ],
    )
    def _triu_gather(adj_hbm, out_hbm, rows_v, seg_v, sems):
        cid = lax.axis_index("c")
        sid = lax.axis_index("s")
        wid = sid * 2 + cid
        a0 = wid * _SEG

        r0 = jnp.int32(_R0S[0])
        r1 = jnp.int32(_R1S[0])
        cls = jnp.int32(_CLS[0])
        for w in range(1, _NW):
            r0 = jnp.where(wid == w, jnp.int32(_R0S[w]), r0)
            r1 = jnp.where(wid == w, jnp.int32(_R1S[w]), r1)
            cls = jnp.where(wid == w, jnp.int32(_CLS[w]), cls)
        nrows = r1 - r0
        wsel = jnp.int32(_WCLS[0])
        for c in range(1, len(_WCLS)):
            wsel = jnp.where(cls == c, jnp.int32(_WCLS[c]), wsel)
        ngroups = (nrows + _KBUF - 1) // _KBUF

        def fire(t, bank):
            rbase = r0 + t * _KBUF
            for b in range(_KBUF):
                i = rbase + b
                slot = bank * _KBUF + b
                live = (t < ngroups) & (i < r1)
                for c, wdt in enumerate(_WCLS):

                    @pl.when(live & (cls == c))
                    def _(i=i, slot=slot, wdt=wdt):
                        pltpu.async_copy(
                            adj_hbm.at[pl.ds(i * _N + (_N - wdt), wdt)],
                            rows_v.at[pl.ds(slot * _ROWP, wdt)],
                            sems[slot])

        def process(t, bank):
            rbase = r0 + t * _KBUF
            for b in range(_KBUF):
                i = rbase + b
                slot = bank * _KBUF + b
                live = (t < ngroups) & (i < r1)
                for c, wdt in enumerate(_WCLS):

                    @pl.when(live & (cls == c))
                    def _(i=i, slot=slot, wdt=wdt):
                        pltpu.make_async_copy(
                            adj_hbm.at[pl.ds(i * _N + (_N - wdt), wdt)],
                            rows_v.at[pl.ds(slot * _ROWP, wdt)],
                            sems[slot]).wait()

                @pl.when(live)
                def _(i=i, slot=slot):
                    off_i = _tri_off(i)
                    skip = jnp.maximum(a0 - off_i, 0)
                    col0 = i + 1 + skip
                    q = off_i + skip - a0
                    length = (_N - 1 - i) - skip
                    nv = (length + 15) >> 4
                    src0 = slot * _ROWP + col0 - _N + wsel

                    def copy16(u, carry):
                        seg_v[pl.ds(q + u * 16, 16)] = (
                            rows_v[pl.ds(src0 + u * 16, 16)])
                        return carry

                    lax.fori_loop(0, nv, copy16, 0)

        fire(0, 0)

        def two_groups(tt, carry):
            t0 = 2 * tt
            fire(t0 + 1, 1)
            process(t0, 0)
            fire(t0 + 2, 0)
            process(t0 + 1, 1)
            return carry

        lax.fori_loop(0, (ngroups + 1) // 2, two_groups, 0)
        pltpu.sync_copy(seg_v.at[pl.ds(0, _SEG)], out_hbm.at[pl.ds(a0, _SEG)])

    return _triu_gather


def kernel(class_labels, z, class_table, Wg1, bg1, Wg2, bg2, We, be, threshold):
    nblk = _N // _BLK
    pair_cols = _M // nblk
    soft_adj, pair_index = pl.pallas_call(
        _gen_body,
        grid=(nblk,),
        out_shape=(jax.ShapeDtypeStruct((_N, _N), jnp.float32),
                   jax.ShapeDtypeStruct((2, _M), jnp.int32)),
        in_specs=[
            pl.BlockSpec(memory_space=pltpu.SMEM),
            pl.BlockSpec(memory_space=pltpu.SMEM),
            pl.BlockSpec(memory_space=pltpu.VMEM),
            pl.BlockSpec(memory_space=pltpu.VMEM),
            pl.BlockSpec(memory_space=pltpu.VMEM),
            pl.BlockSpec(memory_space=pltpu.VMEM),
            pl.BlockSpec(memory_space=pltpu.VMEM),
            pl.BlockSpec(memory_space=pltpu.VMEM),
            pl.BlockSpec(memory_space=pltpu.VMEM),
            pl.BlockSpec(memory_space=pltpu.VMEM),
            pl.BlockSpec((2, pair_cols), lambda i: (0, i)),
        ],
        out_specs=(pl.BlockSpec((_BLK, _N), lambda i: (i, 0)),
                   pl.BlockSpec((2, pair_cols), lambda i: (0, i))),
        scratch_shapes=[pltpu.VMEM((_N, _HIDDEN_DIM), jnp.float32)],
    )(class_labels, jnp.reshape(threshold, (1,)), class_table, z, Wg1,
      bg1[None, :], Wg2, bg2[None, :], We, be[None, :],
      jnp.asarray(_PAIR_NP))

    probs_flat = _make_triu_gather()(jnp.reshape(soft_adj, (_N * _N,)))
    probs = probs_flat[:, None]
    return probs, pair_index, soft_adj
